# baseline (speedup 1.0000x reference)
import jax
import jax.numpy as jnp
from jax.experimental import pallas as pl
from jax.experimental.pallas import tpu as pltpu

B = 1024
C = 1000
D = 64
Q = 65536
TEMP = 0.07
INV_TEMP = 1.0 / TEMP
CON_W = 0.1
C_PAD = 1024
KQ = 4096
NSTEPS = Q // KQ
LOG2E = 1.4426950408889634
NBC = 4
BC = B // NBC
BX = B + 8


def _bpaco_kernel(logits_ref, emb_ref, targets_ref, centers_ref, ci_ref,
                  queue_ref, labels_ref, out_ref, acc_ref, e_ref):
    i = pl.program_id(0)

    @pl.when(i == 0)
    def _init():
        acc_ref[...] = jnp.zeros_like(acc_ref)
        e_ref[B:BX, :] = jnp.ones((8, KQ), jnp.bfloat16)

    emb = emb_ref[...]
    nrm = jnp.sqrt(jnp.sum(emb * emb, axis=1, keepdims=True))
    feats_q = emb * ((INV_TEMP * LOG2E) / jnp.maximum(nrm, 1e-12))

    labels = labels_ref[...]
    iota = jax.lax.broadcasted_iota(jnp.int32, (KQ, C_PAD), 1)
    onehot = (labels == iota).astype(jnp.bfloat16)

    qb = queue_ref[...]
    for c in range(NBC):
        simq = jax.lax.dot_general(feats_q[c * BC:(c + 1) * BC], qb,
                                   (((1,), (1,)), ((), ())),
                                   preferred_element_type=jnp.float32)
        e_ref[c * BC:(c + 1) * BC, :] = jnp.exp2(
            simq - INV_TEMP * LOG2E).astype(jnp.bfloat16)

    acc_ref[...] += jax.lax.dot_general(e_ref[...], onehot,
                                        (((1,), (0,)), ((), ())),
                                        preferred_element_type=jnp.float32)

    @pl.when(i == NSTEPS - 1)
    def _finalize():
        feats = emb * (1.0 / jnp.maximum(nrm, 1e-12))
        sums = acc_ref[0:B, :C]
        counts = acc_ref[B:B + 1, :C]
        queue_logits = jnp.where(
            counts > 0.0,
            jnp.log(jnp.maximum(sums, 1e-30)) + INV_TEMP
            - jnp.log(jnp.maximum(counts, 1.0)),
            0.0)

        centers = jnp.where(ci_ref[...], centers_ref[...], 0.0)
        cn = jnp.sqrt(jnp.sum(centers * centers, axis=1, keepdims=True))
        centers_n = centers / jnp.maximum(cn, 1e-12)
        center_logits = jax.lax.dot_general(
            feats, centers_n, (((1,), (1,)), ((), ())),
            preferred_element_type=jnp.float32) * INV_TEMP

        mx = jnp.maximum(center_logits, queue_logits)
        comp = mx + jnp.log(jnp.exp(center_logits - mx)
                            + jnp.exp(queue_logits - mx))

        tgt = targets_ref[...]
        cls_iota = jax.lax.broadcasted_iota(jnp.int32, (B, C), 1)
        tgt_onehot = (tgt == cls_iota)

        def _ce(a):
            m = jnp.max(a, axis=1, keepdims=True)
            lz = jnp.log(jnp.sum(jnp.exp(a - m), axis=1, keepdims=True)) + m
            picked = jnp.sum(jnp.where(tgt_onehot, a, 0.0), axis=1,
                             keepdims=True)
            return jnp.sum(lz - picked) * (1.0 / B)

        loss = _ce(logits_ref[...]) + CON_W * _ce(comp)
        out_ref[...] = jnp.broadcast_to(loss, (8, 128))


def kernel(logits, embedding, targets, class_centers, center_initialized,
           queue, queue_labels):
    targets_2d = targets.reshape(B, 1)
    labels_2d = queue_labels.reshape(Q, 1)
    ci_2d = center_initialized.reshape(C, 1)

    out = pl.pallas_call(
        _bpaco_kernel,
        grid=(NSTEPS,),
        in_specs=[
            pl.BlockSpec((B, C), lambda i: (0, 0)),
            pl.BlockSpec((B, D), lambda i: (0, 0)),
            pl.BlockSpec((B, 1), lambda i: (0, 0)),
            pl.BlockSpec((C, D), lambda i: (0, 0)),
            pl.BlockSpec((C, 1), lambda i: (0, 0)),
            pl.BlockSpec((KQ, D), lambda i: (i, 0)),
            pl.BlockSpec((KQ, 1), lambda i: (i, 0)),
        ],
        out_specs=pl.BlockSpec((8, 128), lambda i: (0, 0)),
        out_shape=jax.ShapeDtypeStruct((8, 128), jnp.float32),
        scratch_shapes=[
            pltpu.VMEM((BX, C_PAD), jnp.float32),
            pltpu.VMEM((BX, KQ), jnp.bfloat16),
        ],
    )(logits, embedding, targets_2d, class_centers, ci_2d, queue, labels_2d)
    return out[0, 0]


# speedup vs baseline: 1.0044x; 1.0044x over previous
import jax
import jax.numpy as jnp
from jax.experimental import pallas as pl
from jax.experimental.pallas import tpu as pltpu

B = 1024
C = 1000
D = 64
Q = 65536
TEMP = 0.07
INV_TEMP = 1.0 / TEMP
CON_W = 0.1
C_PAD = 1024
KQ = 8192
NSTEPS = Q // KQ
LOG2E = 1.4426950408889634
NBC = 4
BC = B // NBC
BX = B + 8


def _accum_kernel(emb_ref, queue_ref, labels_ref, acc_ref, e_ref):
    i = pl.program_id(0)

    @pl.when(i == 0)
    def _init():
        acc_ref[...] = jnp.zeros_like(acc_ref)
        e_ref[B:BX, :] = jnp.ones((8, KQ), jnp.bfloat16)

    emb = emb_ref[...]
    nrm = jnp.sqrt(jnp.sum(emb * emb, axis=1, keepdims=True))
    feats_q = emb * ((INV_TEMP * LOG2E) / jnp.maximum(nrm, 1e-12))

    labels = labels_ref[...]
    iota = jax.lax.broadcasted_iota(jnp.int32, (KQ, C_PAD), 1)
    onehot = (labels == iota).astype(jnp.bfloat16)

    qb = queue_ref[...]
    for c in range(NBC):
        simq = jax.lax.dot_general(feats_q[c * BC:(c + 1) * BC], qb,
                                   (((1,), (1,)), ((), ())),
                                   preferred_element_type=jnp.float32)
        e_ref[c * BC:(c + 1) * BC, :] = jnp.exp2(
            simq - INV_TEMP * LOG2E).astype(jnp.bfloat16)

    acc_ref[...] += jax.lax.dot_general(e_ref[...], onehot,
                                        (((1,), (0,)), ((), ())),
                                        preferred_element_type=jnp.float32)


def _finalize_kernel(logits_ref, emb_ref, targets_ref, centers_ref, acc_ref,
                     out_ref):
    emb = emb_ref[...]
    nrm = jnp.sqrt(jnp.sum(emb * emb, axis=1, keepdims=True))
    feats = emb / jnp.maximum(nrm, 1e-12)

    sums = acc_ref[0:B, :C]
    counts = acc_ref[B:B + 1, :C]
    queue_logits = jnp.where(
        counts > 0.0,
        jnp.log(jnp.maximum(sums, 1e-30)) + INV_TEMP
        - jnp.log(jnp.maximum(counts, 1.0)),
        0.0)

    centers = centers_ref[...]
    cn = jnp.sqrt(jnp.sum(centers * centers, axis=1, keepdims=True))
    centers_n = centers / jnp.maximum(cn, 1e-12)
    center_logits = jax.lax.dot_general(
        feats, centers_n, (((1,), (1,)), ((), ())),
        preferred_element_type=jnp.float32) * INV_TEMP

    mx = jnp.maximum(center_logits, queue_logits)
    comp = mx + jnp.log(jnp.exp(center_logits - mx) + jnp.exp(queue_logits - mx))

    tgt = targets_ref[...]
    cls_iota = jax.lax.broadcasted_iota(jnp.int32, (B, C), 1)
    tgt_onehot = (tgt == cls_iota)

    def _ce(a):
        m = jnp.max(a, axis=1, keepdims=True)
        lz = jnp.log(jnp.sum(jnp.exp(a - m), axis=1, keepdims=True)) + m
        picked = jnp.sum(jnp.where(tgt_onehot, a, 0.0), axis=1, keepdims=True)
        return jnp.sum(lz - picked) * (1.0 / B)

    loss = _ce(logits_ref[...]) + CON_W * _ce(comp)
    out_ref[...] = jnp.broadcast_to(loss, (8, 128))


def kernel(logits, embedding, targets, class_centers, center_initialized,
           queue, queue_labels):
    centers_masked = class_centers * center_initialized[:, None].astype(jnp.float32)
    targets_2d = targets.reshape(B, 1)
    labels_2d = queue_labels.reshape(Q, 1)

    acc = pl.pallas_call(
        _accum_kernel,
        grid=(NSTEPS,),
        in_specs=[
            pl.BlockSpec((B, D), lambda i: (0, 0)),
            pl.BlockSpec((KQ, D), lambda i: (i, 0)),
            pl.BlockSpec((KQ, 1), lambda i: (i, 0)),
        ],
        out_specs=pl.BlockSpec((BX, C_PAD), lambda i: (0, 0)),
        out_shape=jax.ShapeDtypeStruct((BX, C_PAD), jnp.float32),
        scratch_shapes=[pltpu.VMEM((BX, KQ), jnp.bfloat16)],
    )(embedding, queue, labels_2d)

    out = pl.pallas_call(
        _finalize_kernel,
        in_specs=[
            pl.BlockSpec((B, C), lambda: (0, 0)),
            pl.BlockSpec((B, D), lambda: (0, 0)),
            pl.BlockSpec((B, 1), lambda: (0, 0)),
            pl.BlockSpec((C, D), lambda: (0, 0)),
            pl.BlockSpec((BX, C_PAD), lambda: (0, 0)),
        ],
        out_specs=pl.BlockSpec((8, 128), lambda: (0, 0)),
        out_shape=jax.ShapeDtypeStruct((8, 128), jnp.float32),
    )(logits, embedding, targets_2d, centers_masked, acc)
    return out[0, 0]
